# baseline (speedup 1.0000x reference)
import functools

import jax
import jax.numpy as jnp
from jax import lax
from jax.experimental import pallas as pl
from jax.experimental.pallas import tpu as pltpu
from jax.experimental.pallas import tpu_sc as plsc
import numpy as np

_SIZE = 32000
_PAD = 0
_SMOOTHING = 0.1
_CONFIDENCE = 1.0 - _SMOOTHING
_N_TOK = 4096
_LATENT = 512

_EPS = np.float32(_SMOOTHING / (_SIZE - 2))
_YLOGY = np.float32(
    (_SIZE - 2) * float(_EPS) * np.log(float(_EPS))
    + _CONFIDENCE * np.log(_CONFIDENCE)
)

_SC_NC = 2
_SC_NS = 16
_SC_L = 16
_SC_NW = _SC_NC * _SC_NS

_SC_XROWS = 512
_TC_ROWS = _N_TOK - _SC_XROWS
_SC_RPW = _SC_XROWS // _SC_NW
_SC_CCH = 6400
_SC_U = 400
_SC_NCH = _SIZE // _SC_CCH
_SC_CH = 32

_RBLK = 512
_CBLK = 1280
_RGRID = _TC_ROWS // _RBLK
_CGRID = _SIZE // _CBLK
_ACCW = 128
_NGRP = _CBLK // _ACCW


def _loss_body(x_ref, tgt_ref, accs_ref, acc128_ref, tacc128_ref):
    i = pl.program_id(0)
    j = pl.program_id(1)

    @pl.when((i == 0) & (j == 0))
    def _init():
        accs_ref[0, 0] = 0.0
        accs_ref[0, 1] = 0.0
        accs_ref[0, 2] = 0.0
        accs_ref[0, 3] = 0.0

    xb = x_ref[...]
    tgt = tgt_ref[0, 0, :]

    col = j * _CBLK + lax.broadcasted_iota(jnp.int32, (_RBLK, _CBLK), 1)
    hit = col == tgt[:, None]
    rs = xb[:, 0:_ACCW]
    ts = jnp.where(hit[:, 0:_ACCW], xb[:, 0:_ACCW], 0.0)
    for g in range(1, _NGRP):
        sl = slice(g * _ACCW, (g + 1) * _ACCW)
        rs = rs + xb[:, sl]
        ts = ts + jnp.where(hit[:, sl], xb[:, sl], 0.0)

    @pl.when(j == 0)
    def _first_colblock():
        acc128_ref[...] = rs
        tacc128_ref[...] = ts
        w = (tgt != _PAD).astype(jnp.float32)
        accs_ref[0, 1] = accs_ref[0, 1] + jnp.sum(xb[:, _PAD] * w)
        accs_ref[0, 2] = accs_ref[0, 2] + jnp.sum(w)

    @pl.when(j > 0)
    def _accum():
        acc128_ref[...] = acc128_ref[...] + rs
        tacc128_ref[...] = tacc128_ref[...] + ts

    @pl.when(j == _CGRID - 1)
    def _last_colblock():
        w = (tgt != _PAD).astype(jnp.float32)
        rowsum = jnp.sum(acc128_ref[...], axis=1)
        tval = jnp.sum(tacc128_ref[...], axis=1)
        accs_ref[0, 0] = accs_ref[0, 0] + jnp.sum(rowsum * w)
        accs_ref[0, 3] = accs_ref[0, 3] + jnp.sum(tval * w)


_sc_mesh = plsc.VectorSubcoreMesh(core_axis_name="c", subcore_axis_name="s")


@functools.partial(
    pl.kernel,
    mesh=_sc_mesh,
    out_type=[
        jax.ShapeDtypeStruct((_SC_NW, _SC_RPW, _SC_L), jnp.float32),
        jax.ShapeDtypeStruct((_SC_NW, _SC_L), jnp.float32),
        jax.ShapeDtypeStruct((_SC_NW, _SC_L), jnp.float32),
        jax.ShapeDtypeStruct((_SC_NW, _SC_L), jnp.float32),
        jax.ShapeDtypeStruct((_SC_NW, _SC_L), jnp.float32),
    ],
    scratch_types=[
        pltpu.VMEM((_SC_CCH,), jnp.float32),
        pltpu.VMEM((_SC_CCH,), jnp.float32),
        pltpu.VMEM((_SC_L,), jnp.int32),
        pltpu.VMEM((_SC_RPW, _SC_L), jnp.float32),
        pltpu.VMEM((_SC_L,), jnp.float32),
        pltpu.VMEM((_SC_CH, _LATENT), jnp.float32),
        pltpu.VMEM((_SC_CH, _LATENT), jnp.float32),
        pltpu.SemaphoreType.DMA,
        pltpu.SemaphoreType.DMA,
    ],
)
def _sc_part(x_hbm, tgt_hbm, mu_hbm, lv_hbm,
             rs_out, tgt_out, c0_out, cnt_out, kl_out,
             xbuf0, xbuf1, tgt_v, rs_stage, vec_stage, mu_v, lv_v,
             sem0, sem1):
    wid = lax.axis_index("s") * _SC_NC + lax.axis_index("c")
    row0 = _TC_ROWS + wid * _SC_RPW
    pltpu.sync_copy(tgt_hbm.at[pl.ds(row0, _SC_RPW)], tgt_v)

    tgt16 = tgt_v[...]
    iota16 = lax.iota(jnp.int32, _SC_L)
    wmask = tgt16 != _PAD
    zero16 = jnp.zeros((_SC_L,), jnp.float32)

    bufs = (xbuf0, xbuf1)
    sems = (sem0, sem1)
    nq = _SC_RPW * _SC_NCH

    def _start(q):
        rr, c = divmod(q, _SC_NCH)
        return pltpu.async_copy(
            x_hbm.at[row0 + rr, pl.ds(c * _SC_CCH, _SC_CCH)],
            bufs[q % 2], sems[q % 2])

    tacc = zero16
    c0acc = zero16
    handle = _start(0)
    racc0 = zero16
    racc1 = zero16
    for q in range(nq):
        rr, c = divmod(q, _SC_NCH)
        handle.wait()
        if q + 1 < nq:
            handle = _start(q + 1)
        buf = bufs[q % 2]

        def kbody(k, ab):
            a0, a1 = ab
            k0 = pl.multiple_of(k * _SC_U, _SC_L)
            for u in range(0, _SC_U // _SC_L, 2):
                a0 = a0 + buf[pl.ds(k0 + u * _SC_L, _SC_L)]
            for u in range(1, _SC_U // _SC_L, 2):
                a1 = a1 + buf[pl.ds(k0 + u * _SC_L, _SC_L)]
            return (a0, a1)

        racc0, racc1 = lax.fori_loop(0, _SC_CCH // _SC_U, kbody,
                                     (racc0, racc1))

        t_here = tgt16[rr]
        cs = t_here - c * _SC_CCH
        in_chunk = (cs >= 0) & (cs < _SC_CCH) & (t_here != _PAD)
        fsc = jnp.where(in_chunk, jnp.float32(1.0), jnp.float32(0.0))
        woff = jnp.clip((cs >> 4) << 4, 0, _SC_CCH - _SC_L)
        v = buf[pl.ds(woff, _SC_L)]
        lmask = jnp.where(iota16 == (cs & 15), jnp.float32(1.0),
                          jnp.float32(0.0))
        tacc = tacc + v * (lmask * fsc)
        if c == 0:
            fw = jnp.where(t_here != _PAD, jnp.float32(1.0), jnp.float32(0.0))
            v0 = buf[pl.ds(0, _SC_L)]
            lmask0 = jnp.where(iota16 == 0, jnp.float32(1.0),
                               jnp.float32(0.0))
            c0acc = c0acc + v0 * (lmask0 * fw)
        if c == _SC_NCH - 1:
            rs_stage[rr, :] = racc0 + racc1
            racc0 = zero16
            racc1 = zero16

    pltpu.sync_copy(rs_stage, rs_out.at[wid])
    vec_stage[...] = tacc
    pltpu.sync_copy(vec_stage, tgt_out.at[wid])
    vec_stage[...] = c0acc
    pltpu.sync_copy(vec_stage, c0_out.at[wid])
    vec_stage[...] = jnp.where(wmask, 1.0, 0.0)
    pltpu.sync_copy(vec_stage, cnt_out.at[wid])

    klbase = wid * (_N_TOK // _SC_NW)

    def chunk_body(cc, acc):
        r0 = pl.multiple_of(klbase + cc * _SC_CH, _SC_CH)
        pltpu.sync_copy(mu_hbm.at[pl.ds(r0, _SC_CH)], mu_v)
        pltpu.sync_copy(lv_hbm.at[pl.ds(r0, _SC_CH)], lv_v)

        def row_body(r, acc2):
            for k in range(_LATENT // _SC_L):
                mv = mu_v[r, pl.ds(k * _SC_L, _SC_L)]
                lv = lv_v[r, pl.ds(k * _SC_L, _SC_L)]
                acc2 = acc2 + (1.0 + lv - mv * mv - jnp.exp(lv))
            return acc2

        return lax.fori_loop(0, _SC_CH, row_body, acc)

    klacc = lax.fori_loop(0, (_N_TOK // _SC_NW) // _SC_CH, chunk_body, zero16)
    vec_stage[...] = klacc
    pltpu.sync_copy(vec_stage, kl_out.at[wid])


def _combine_body(accs_ref, rs_ref, tgtt_ref, tgtp_ref, c0p_ref, cntp_ref,
                  klp_ref, rec_ref, kl_ref):
    w = (tgtt_ref[0, :] != _PAD).astype(jnp.float32)
    rowsum_tail = jnp.sum(rs_ref[...], axis=1)
    s_all = accs_ref[0, 0] + jnp.sum(rowsum_tail * w)
    s_c0 = accs_ref[0, 1] + jnp.sum(c0p_ref[...])
    n = accs_ref[0, 2] + jnp.sum(cntp_ref[...])
    s_tgt = accs_ref[0, 3] + jnp.sum(tgtp_ref[...])
    rec_ref[0, 0] = (
        n * _YLOGY - _EPS * (s_all - s_c0)
        - (np.float32(_CONFIDENCE) - _EPS) * s_tgt
    )
    kl_ref[0, 0] = -0.5 * jnp.sum(klp_ref[...]) / np.float32(_N_TOK * _LATENT)


@jax.jit
def kernel(x, target, mu, logvar):
    tgt3 = target[:_TC_ROWS].reshape(_RGRID, 1, _RBLK)
    rs_p, tgt_p, c0_p, cnt_p, kl_p = _sc_part(x, target, mu, logvar)
    (accs,) = pl.pallas_call(
        _loss_body,
        grid=(_RGRID, _CGRID),
        in_specs=[
            pl.BlockSpec((_RBLK, _CBLK), lambda i, j: (i, j)),
            pl.BlockSpec((1, 1, _RBLK), lambda i, j: (i, 0, 0)),
        ],
        out_specs=[
            pl.BlockSpec(memory_space=pltpu.SMEM),
        ],
        out_shape=[
            jax.ShapeDtypeStruct((1, 4), jnp.float32),
        ],
        scratch_shapes=[
            pltpu.VMEM((_RBLK, _ACCW), jnp.float32),
            pltpu.VMEM((_RBLK, _ACCW), jnp.float32),
        ],
    )(x, tgt3)
    rec, kl = pl.pallas_call(
        _combine_body,
        in_specs=[
            pl.BlockSpec(memory_space=pltpu.SMEM),
            pl.BlockSpec((_SC_XROWS, _SC_L), lambda: (0, 0)),
            pl.BlockSpec((1, _SC_XROWS), lambda: (0, 0)),
            pl.BlockSpec((_SC_NW, _SC_L), lambda: (0, 0)),
            pl.BlockSpec((_SC_NW, _SC_L), lambda: (0, 0)),
            pl.BlockSpec((_SC_NW, _SC_L), lambda: (0, 0)),
            pl.BlockSpec((_SC_NW, _SC_L), lambda: (0, 0)),
        ],
        out_specs=[
            pl.BlockSpec(memory_space=pltpu.SMEM),
            pl.BlockSpec(memory_space=pltpu.SMEM),
        ],
        out_shape=[
            jax.ShapeDtypeStruct((1, 1), jnp.float32),
            jax.ShapeDtypeStruct((1, 1), jnp.float32),
        ],
    )(accs, rs_p.reshape(_SC_XROWS, _SC_L),
      target[_TC_ROWS:].reshape(1, _SC_XROWS), tgt_p, c0_p, cnt_p, kl_p)
    return (rec[0, 0], kl[0, 0])


# speedup vs baseline: 1.1842x; 1.1842x over previous
import functools

import jax
import jax.numpy as jnp
from jax import lax
from jax.experimental import pallas as pl
from jax.experimental.pallas import tpu as pltpu
from jax.experimental.pallas import tpu_sc as plsc
import numpy as np

_SIZE = 32000
_PAD = 0
_SMOOTHING = 0.1
_CONFIDENCE = 1.0 - _SMOOTHING
_N_TOK = 4096
_LATENT = 512

_EPS = np.float32(_SMOOTHING / (_SIZE - 2))
_YLOGY = np.float32(
    (_SIZE - 2) * float(_EPS) * np.log(float(_EPS))
    + _CONFIDENCE * np.log(_CONFIDENCE)
)

_RBLK = 1024
_CBLK = 1280
_RGRID = _N_TOK // _RBLK
_CGRID = _SIZE // _CBLK
_ACCW = 128
_NGRP = _CBLK // _ACCW

_SC_NC = 2
_SC_NS = 16
_SC_L = 16
_SC_NW = _SC_NC * _SC_NS
_SC_ROWS = _N_TOK // _SC_NW
_SC_CH = 32


def _loss_body(x_ref, tgt_ref, rec_ref, acc_ref, acc128_ref, tacc128_ref):
    i = pl.program_id(0)
    j = pl.program_id(1)

    @pl.when((i == 0) & (j == 0))
    def _init():
        acc_ref[0] = 0.0
        acc_ref[1] = 0.0
        acc_ref[2] = 0.0
        acc_ref[3] = 0.0
        rec_ref[0, 0] = 0.0

    xb = x_ref[...]
    tgt = tgt_ref[0, 0, :]

    col = j * _CBLK + lax.broadcasted_iota(jnp.int32, (_RBLK, _CBLK), 1)
    hit = col == tgt[:, None]
    rs = xb[:, 0:_ACCW]
    ts = jnp.where(hit[:, 0:_ACCW], xb[:, 0:_ACCW], 0.0)
    for g in range(1, _NGRP):
        sl = slice(g * _ACCW, (g + 1) * _ACCW)
        rs = rs + xb[:, sl]
        ts = ts + jnp.where(hit[:, sl], xb[:, sl], 0.0)

    @pl.when(j == 0)
    def _first_colblock():
        acc128_ref[...] = rs
        tacc128_ref[...] = ts
        w = (tgt != _PAD).astype(jnp.float32)
        acc_ref[1] = acc_ref[1] + jnp.sum(xb[:, _PAD] * w)
        acc_ref[2] = acc_ref[2] + jnp.sum(w)

    @pl.when(j > 0)
    def _accum():
        acc128_ref[...] = acc128_ref[...] + rs
        tacc128_ref[...] = tacc128_ref[...] + ts

    @pl.when(j == _CGRID - 1)
    def _last_colblock():
        w = (tgt != _PAD).astype(jnp.float32)
        rowsum = jnp.sum(acc128_ref[...], axis=1)
        tval = jnp.sum(tacc128_ref[...], axis=1)
        acc_ref[0] = acc_ref[0] + jnp.sum(rowsum * w)
        acc_ref[3] = acc_ref[3] + jnp.sum(tval * w)

    @pl.when((i == _RGRID - 1) & (j == _CGRID - 1))
    def _finalize():
        rec_ref[0, 0] = (
            acc_ref[2] * _YLOGY
            - _EPS * (acc_ref[0] - acc_ref[1])
            - (np.float32(_CONFIDENCE) - _EPS) * acc_ref[3]
        )


_sc_mesh = plsc.VectorSubcoreMesh(core_axis_name="c", subcore_axis_name="s")


@functools.partial(
    pl.kernel,
    mesh=_sc_mesh,
    out_type=jax.ShapeDtypeStruct((_SC_NW, _SC_L), jnp.float32),
    scratch_types=[
        pltpu.VMEM((_SC_CH, _LATENT), jnp.float32),
        pltpu.VMEM((_SC_CH, _LATENT), jnp.float32),
        pltpu.VMEM((_SC_L,), jnp.float32),
    ],
)
def _sc_kl(mu_hbm, lv_hbm, out_hbm, mu_v, lv_v, part_v):
    wid = lax.axis_index("s") * _SC_NC + lax.axis_index("c")
    base = wid * _SC_ROWS

    def chunk_body(c, acc):
        row0 = pl.multiple_of(base + c * _SC_CH, _SC_CH)
        pltpu.sync_copy(mu_hbm.at[pl.ds(row0, _SC_CH)], mu_v)
        pltpu.sync_copy(lv_hbm.at[pl.ds(row0, _SC_CH)], lv_v)

        def row_body(r, acc2):
            for k in range(_LATENT // _SC_L):
                m = mu_v[r, pl.ds(k * _SC_L, _SC_L)]
                l = lv_v[r, pl.ds(k * _SC_L, _SC_L)]
                acc2 = acc2 + (1.0 + l - m * m - jnp.exp(l))
            return acc2

        return lax.fori_loop(0, _SC_CH, row_body, acc)

    acc = lax.fori_loop(0, _SC_ROWS // _SC_CH, chunk_body,
                        jnp.zeros((_SC_L,), jnp.float32))
    part_v[...] = acc
    pltpu.sync_copy(part_v, out_hbm.at[wid])


@jax.jit
def kernel(x, target, mu, logvar):
    tgt3 = target.reshape(_RGRID, 1, _RBLK)
    kl_parts = _sc_kl(mu, logvar)
    (rec,) = pl.pallas_call(
        _loss_body,
        grid=(_RGRID, _CGRID),
        in_specs=[
            pl.BlockSpec((_RBLK, _CBLK), lambda i, j: (i, j)),
            pl.BlockSpec((1, 1, _RBLK), lambda i, j: (i, 0, 0)),
        ],
        out_specs=[
            pl.BlockSpec(memory_space=pltpu.SMEM),
        ],
        out_shape=[
            jax.ShapeDtypeStruct((1, 1), jnp.float32),
        ],
        scratch_shapes=[
            pltpu.SMEM((4,), jnp.float32),
            pltpu.VMEM((_RBLK, _ACCW), jnp.float32),
            pltpu.VMEM((_RBLK, _ACCW), jnp.float32),
        ],
    )(x, tgt3)
    kl = -0.5 * jnp.sum(kl_parts) / np.float32(_N_TOK * _LATENT)
    return (rec[0, 0], kl)


# speedup vs baseline: 1.3204x; 1.1150x over previous
import functools

import jax
import jax.numpy as jnp
from jax import lax
from jax.experimental import pallas as pl
from jax.experimental.pallas import tpu as pltpu
from jax.experimental.pallas import tpu_sc as plsc
import numpy as np

_SIZE = 32000
_PAD = 0
_SMOOTHING = 0.1
_CONFIDENCE = 1.0 - _SMOOTHING
_N_TOK = 4096
_LATENT = 512

_EPS = np.float32(_SMOOTHING / (_SIZE - 2))
_YLOGY = np.float32(
    (_SIZE - 2) * float(_EPS) * np.log(float(_EPS))
    + _CONFIDENCE * np.log(_CONFIDENCE)
)

_RBLK = 2048
_CBLK = 1280
_RGRID = _N_TOK // _RBLK
_CGRID = _SIZE // _CBLK
_ACCW = 128
_NGRP = _CBLK // _ACCW

_SC_NC = 2
_SC_NS = 16
_SC_L = 16
_SC_NW = _SC_NC * _SC_NS
_SC_ROWS = _N_TOK // _SC_NW
_SC_CH = 32


def _loss_body(x_ref, tgt_ref, rec_ref, acc_ref, acc128_ref, tacc128_ref):
    i = pl.program_id(0)
    j = pl.program_id(1)

    @pl.when((i == 0) & (j == 0))
    def _init():
        acc_ref[0] = 0.0
        acc_ref[1] = 0.0
        acc_ref[2] = 0.0
        acc_ref[3] = 0.0
        rec_ref[0, 0] = 0.0

    xb = x_ref[...]
    tgt = tgt_ref[0, 0, :]

    col = j * _CBLK + lax.broadcasted_iota(jnp.int32, (_RBLK, _CBLK), 1)
    hit = col == tgt[:, None]
    rs = xb[:, 0:_ACCW]
    ts = jnp.where(hit[:, 0:_ACCW], xb[:, 0:_ACCW], 0.0)
    for g in range(1, _NGRP):
        sl = slice(g * _ACCW, (g + 1) * _ACCW)
        rs = rs + xb[:, sl]
        ts = ts + jnp.where(hit[:, sl], xb[:, sl], 0.0)

    @pl.when(j == 0)
    def _first_colblock():
        acc128_ref[...] = rs
        tacc128_ref[...] = ts
        w = (tgt != _PAD).astype(jnp.float32)
        acc_ref[1] = acc_ref[1] + jnp.sum(xb[:, _PAD] * w)
        acc_ref[2] = acc_ref[2] + jnp.sum(w)

    @pl.when(j > 0)
    def _accum():
        acc128_ref[...] = acc128_ref[...] + rs
        tacc128_ref[...] = tacc128_ref[...] + ts

    @pl.when(j == _CGRID - 1)
    def _last_colblock():
        w = (tgt != _PAD).astype(jnp.float32)
        rowsum = jnp.sum(acc128_ref[...], axis=1)
        tval = jnp.sum(tacc128_ref[...], axis=1)
        acc_ref[0] = acc_ref[0] + jnp.sum(rowsum * w)
        acc_ref[3] = acc_ref[3] + jnp.sum(tval * w)

    @pl.when((i == _RGRID - 1) & (j == _CGRID - 1))
    def _finalize():
        rec_ref[0, 0] = (
            acc_ref[2] * _YLOGY
            - _EPS * (acc_ref[0] - acc_ref[1])
            - (np.float32(_CONFIDENCE) - _EPS) * acc_ref[3]
        )


_sc_mesh = plsc.VectorSubcoreMesh(core_axis_name="c", subcore_axis_name="s")


@functools.partial(
    pl.kernel,
    mesh=_sc_mesh,
    out_type=jax.ShapeDtypeStruct((_SC_NW, _SC_L), jnp.float32),
    scratch_types=[
        pltpu.VMEM((_SC_CH, _LATENT), jnp.float32),
        pltpu.VMEM((_SC_CH, _LATENT), jnp.float32),
        pltpu.VMEM((_SC_L,), jnp.float32),
    ],
)
def _sc_kl(mu_hbm, lv_hbm, out_hbm, mu_v, lv_v, part_v):
    wid = lax.axis_index("s") * _SC_NC + lax.axis_index("c")
    base = wid * _SC_ROWS

    def chunk_body(c, acc):
        row0 = pl.multiple_of(base + c * _SC_CH, _SC_CH)
        pltpu.sync_copy(mu_hbm.at[pl.ds(row0, _SC_CH)], mu_v)
        pltpu.sync_copy(lv_hbm.at[pl.ds(row0, _SC_CH)], lv_v)

        def row_body(r, acc2):
            for k in range(_LATENT // _SC_L):
                m = mu_v[r, pl.ds(k * _SC_L, _SC_L)]
                l = lv_v[r, pl.ds(k * _SC_L, _SC_L)]
                acc2 = acc2 + (1.0 + l - m * m - jnp.exp(l))
            return acc2

        return lax.fori_loop(0, _SC_CH, row_body, acc)

    acc = lax.fori_loop(0, _SC_ROWS // _SC_CH, chunk_body,
                        jnp.zeros((_SC_L,), jnp.float32))
    part_v[...] = acc
    pltpu.sync_copy(part_v, out_hbm.at[wid])


@jax.jit
def kernel(x, target, mu, logvar):
    tgt3 = target.reshape(_RGRID, 1, _RBLK)
    kl_parts = _sc_kl(mu, logvar)
    (rec,) = pl.pallas_call(
        _loss_body,
        grid=(_RGRID, _CGRID),
        in_specs=[
            pl.BlockSpec((_RBLK, _CBLK), lambda i, j: (i, j)),
            pl.BlockSpec((1, 1, _RBLK), lambda i, j: (i, 0, 0)),
        ],
        out_specs=[
            pl.BlockSpec(memory_space=pltpu.SMEM),
        ],
        out_shape=[
            jax.ShapeDtypeStruct((1, 1), jnp.float32),
        ],
        scratch_shapes=[
            pltpu.SMEM((4,), jnp.float32),
            pltpu.VMEM((_RBLK, _ACCW), jnp.float32),
            pltpu.VMEM((_RBLK, _ACCW), jnp.float32),
        ],
    )(x, tgt3)
    kl = -0.5 * jnp.sum(kl_parts) / np.float32(_N_TOK * _LATENT)
    return (rec[0, 0], kl)
